# Initial kernel scaffold; baseline (speedup 1.0000x reference)
import jax
import jax.numpy as jnp
from jax.experimental import pallas as pl


def kernel(x, edge_index, W, b):
    raise NotImplementedError("write your pallas kernel here")


# speedup vs baseline: 4.8080x; 4.8080x over previous
import functools

import jax
import jax.numpy as jnp
from jax import lax
from jax.experimental import pallas as pl
from jax.experimental.pallas import tpu as pltpu
from jax.experimental.pallas import tpu_sc as plsc

_K = 1.0

_NC, _NS = 2, 16
_NW = _NC * _NS
_CHUNK = 128
_ROW_BLK = 1000


def _proj_body(x_ref, wt_ref, b_ref, h_ref):
    y = jnp.dot(x_ref[...], wt_ref[...], preferred_element_type=jnp.float32)
    y = y + b_ref[...]
    h0 = jnp.sqrt(_K + jnp.sum(y * y, axis=1, keepdims=True))
    col = lax.broadcasted_iota(jnp.int32, y.shape, 1)
    h_ref[...] = jnp.where(col == 0, h0, y)


def _norm_body(mu0_ref, mu1_ref, out_ref):
    mu = mu0_ref[...] + mu1_ref[...]
    t = mu[:, 0:1]
    ss = jnp.sum(mu * mu, axis=1, keepdims=True)
    denom = jnp.sqrt(jnp.maximum(2.0 * t * t - ss, 1e-8))
    hc = (jnp.sqrt(_K) / denom) * mu
    s = jnp.maximum(hc, 0.0)
    s0 = s[:, 0:1]
    o0 = jnp.sqrt(_K + jnp.sum(s * s, axis=1, keepdims=True) - s0 * s0)
    col = lax.broadcasted_iota(jnp.int32, s.shape, 1)
    out_ref[...] = jnp.where(col == 0, o0, s)


@functools.lru_cache(maxsize=None)
def _make_scatter(n_rows, k_chunks, d):
    rows_per_tile = n_rows // _NS
    mesh = plsc.VectorSubcoreMesh(core_axis_name="c", subcore_axis_name="s")

    @functools.partial(
        pl.kernel,
        out_type=jax.ShapeDtypeStruct((_NC, n_rows, d), jnp.float32),
        mesh=mesh,
        scratch_types=[
            pltpu.VMEM((k_chunks, _CHUNK), jnp.int32),
            pltpu.VMEM((k_chunks, _CHUNK), jnp.int32),
            pltpu.VMEM((_CHUNK, d), jnp.float32),
            pltpu.VMEM_SHARED((n_rows, d), jnp.float32),
            pltpu.SemaphoreType.DMA,
        ],
    )
    def scatter_k(h_hbm, src_hbm, dst_hbm, zero_hbm, out_hbm,
                  src_v, dst_v, rows_v, mu_sh, sem):
        cid = lax.axis_index("c")
        sid = lax.axis_index("s")
        wid = cid * _NS + sid
        r0 = sid * rows_per_tile
        pltpu.sync_copy(zero_hbm.at[pl.ds(r0, rows_per_tile)],
                        mu_sh.at[pl.ds(r0, rows_per_tile)])
        pltpu.sync_copy(src_hbm.at[wid], src_v)
        pltpu.sync_copy(dst_hbm.at[wid], dst_v)
        plsc.subcore_barrier()

        def step(j, carry):
            pltpu.async_copy(h_hbm.at[src_v.at[j]], rows_v, sem).wait()
            pltpu.sync_copy(rows_v, mu_sh.at[dst_v.at[j]], add=True)
            return carry

        lax.fori_loop(0, k_chunks, step, 0)
        plsc.subcore_barrier()
        pltpu.sync_copy(mu_sh.at[pl.ds(r0, rows_per_tile)],
                        out_hbm.at[cid, pl.ds(r0, rows_per_tile)])

    return scatter_k


def kernel(x, edge_index, W, b):
    n, d = x.shape
    e = edge_index.shape[1]

    wt = jnp.concatenate([jnp.zeros((d, 1), x.dtype), W.T], axis=1)
    bp = jnp.concatenate([jnp.zeros((1,), x.dtype), b]).reshape(1, d)
    grid1 = n // _ROW_BLK
    h = pl.pallas_call(
        _proj_body,
        grid=(grid1,),
        in_specs=[
            pl.BlockSpec((_ROW_BLK, d), lambda i: (i, 0)),
            pl.BlockSpec((d, d), lambda i: (0, 0)),
            pl.BlockSpec((1, d), lambda i: (0, 0)),
        ],
        out_specs=pl.BlockSpec((_ROW_BLK, d), lambda i: (i, 0)),
        out_shape=jax.ShapeDtypeStruct((n, d), jnp.float32),
    )(x, wt, bp)

    per_tile = -(-e // _NW)
    k_chunks = -(-per_tile // _CHUNK)
    e_pad = _NW * k_chunks * _CHUNK
    n_rows = -(-(n + 1) // (_NS * 8)) * (_NS * 8)
    src = jnp.concatenate(
        [edge_index[0], jnp.zeros((e_pad - e,), jnp.int32)]).reshape(_NW, k_chunks, _CHUNK)
    dst = jnp.concatenate(
        [edge_index[1], jnp.full((e_pad - e,), n, jnp.int32)]).reshape(_NW, k_chunks, _CHUNK)
    zeros = jnp.zeros((n_rows, d), jnp.float32)
    parts = _make_scatter(n_rows, k_chunks, d)(h, src, dst, zeros)
    mu0 = parts[0, :n]
    mu1 = parts[1, :n]

    out = pl.pallas_call(
        _norm_body,
        grid=(grid1,),
        in_specs=[
            pl.BlockSpec((_ROW_BLK, d), lambda i: (i, 0)),
            pl.BlockSpec((_ROW_BLK, d), lambda i: (i, 0)),
        ],
        out_specs=pl.BlockSpec((_ROW_BLK, d), lambda i: (i, 0)),
        out_shape=jax.ShapeDtypeStruct((n, d), jnp.float32),
    )(mu0, mu1)
    return (out, edge_index)
